# baseline (speedup 1.0000x reference)
import functools

import jax
import jax.numpy as jnp
import numpy as np
from jax import lax
from jax.experimental import pallas as pl
from jax.experimental.pallas import tpu as pltpu
from jax.experimental.pallas import tpu_sc as plsc

N = 10000
E = 160000
C = 128
OUT = 128
B = 2
HORIZON = 3

NTILES = 16
CHUNK = 64
NCH = 160
EPT = NCH * CHUNK
EPAD = NTILES * EPT
NPAD = 10240
RPT = NPAD // NTILES
HALF = NCH // 4
NBUF = 4

ROWS = B * N
BLK = 2000
NPB = N // BLK


def _seg_body(PP, u_hbm, src_hbm, dst_hbm, zrow_hbm, out_hbm,
              acc, srcv, dstv, bufs, gsems, ssems):
    c = lax.axis_index("c")
    w = lax.axis_index("s")

    def wait_g(u, k, b):
        pltpu.make_async_copy(u.at[srcv.at[k]], bufs[b], gsems[b]).wait()

    def start_g(u, k, b):
        pltpu.async_copy(u.at[srcv.at[k]], bufs[b], gsems[b])

    def start_s(k, b):
        pltpu.async_copy(bufs[b], acc.at[dstv.at[k]], ssems[b], add=True)

    def wait_s(k, b):
        pltpu.make_async_copy(bufs[b], acc.at[dstv.at[k]], ssems[b]).wait()

    for j in range(PP):
        plane = c * PP + j
        u = u_hbm.at[plane]
        pltpu.sync_copy(zrow_hbm, bufs[0])
        for q in range(RPT // CHUNK):
            pltpu.sync_copy(bufs[0], acc.at[pl.ds(w * RPT + q * CHUNK, CHUNK)])
        plsc.subcore_barrier()
        for h in range(NCH // HALF):
            pltpu.sync_copy(src_hbm.at[w].at[pl.ds(h * HALF, HALF)], srcv)
            pltpu.sync_copy(dst_hbm.at[w].at[pl.ds(h * HALF, HALF)], dstv)
            for b in range(NBUF):
                start_g(u, b, b)
            for k in (0, 1):
                wait_g(u, k, k)
                start_s(k, k)
            def body(i, _):
                for t in range(NBUF):
                    k = NBUF * i + 2 + t
                    b = (2 + t) % NBUF
                    wait_g(u, k, b)
                    start_s(k, b)
                    wait_s(k - 2, (b + 2) % NBUF)
                    start_g(u, k + 2, (b + 2) % NBUF)
                return 0

            lax.fori_loop(0, (HALF - 4) // NBUF, body, 0)
            for k in (HALF - 2, HALF - 1):
                b = k % NBUF
                wait_g(u, k, b)
                start_s(k, b)
            for k in range(HALF - 4, HALF):
                wait_s(k, k % NBUF)
        plsc.subcore_barrier()
        pltpu.sync_copy(acc.at[pl.ds(w * RPT, RPT)],
                        out_hbm.at[plane].at[pl.ds(w * RPT, RPT)])


@functools.lru_cache(maxsize=None)
def _make_seg(P):
    PP = P // 2
    mesh = plsc.VectorSubcoreMesh(core_axis_name="c", subcore_axis_name="s")

    @functools.partial(
        pl.kernel, mesh=mesh,
        out_type=jax.ShapeDtypeStruct((P, NPAD, C), jnp.float32),
        scratch_types=(
            [pltpu.VMEM_SHARED((NPAD, C), jnp.float32)]
            + [pltpu.VMEM((HALF, CHUNK), jnp.int32)] * 2
            + [pltpu.VMEM((CHUNK, C), jnp.float32)] * NBUF
            + [pltpu.SemaphoreType.DMA] * (2 * NBUF)
        ),
    )
    def seg(u_hbm, src_hbm, dst_hbm, zrow_hbm, out_hbm, *rest):
        acc, srcv, dstv = rest[0], rest[1], rest[2]
        bufs = rest[3:3 + NBUF]
        gsems = rest[3 + NBUF:3 + 2 * NBUF]
        ssems = rest[3 + 2 * NBUF:3 + 3 * NBUF]
        _seg_body(PP, u_hbm, src_hbm, dst_hbm, zrow_hbm, out_hbm,
                  acc, srcv, dstv, bufs, gsems, ssems)

    return seg


def _dg(d_ref):
    return 1.0 / jnp.maximum(d_ref[:, 0:1], 1.0)


def _row_spec():
    return pl.BlockSpec((BLK, C), lambda i: (i, 0))


def _deg_spec():
    return pl.BlockSpec((BLK, 16), lambda i: (i % NPB, 0))


def _w_spec(shape):
    return pl.BlockSpec(shape, lambda i: (0, 0))


def _out_rows(n):
    return [jax.ShapeDtypeStruct((ROWS, C), jnp.float32) for _ in range(n)]


@functools.lru_cache(maxsize=None)
def _make_tc1(has_x):
    def kern(*refs):
        if has_x:
            (x_ref, h_ref, sx_ref, sh_ref, d_ref, wx_ref, wh_ref, b_ref,
             p_ref, z_ref, r_ref) = refs
        else:
            (h_ref, sh_ref, d_ref, wh_ref, b_ref, z_ref, r_ref) = refs
        dg = _dg(d_ref)
        q = h_ref[...] + dg * sh_ref[...]
        acc = jnp.dot(q, wh_ref[...], preferred_element_type=jnp.float32)
        if has_x:
            p = x_ref[...] + dg * sx_ref[...]
            acc += jnp.dot(p, wx_ref[...], preferred_element_type=jnp.float32)
            p_ref[...] = p
        zr = jax.nn.sigmoid(acc + b_ref[...])
        z_ref[...] = zr[:, :C]
        r_ref[...] = zr[:, C:] * h_ref[...]

    n_in = 8 if has_x else 5
    in_specs = ([_row_spec()] * (4 if has_x else 2) + [_deg_spec()]
                + [_w_spec((C, 2 * C))] * (2 if has_x else 1)
                + [_w_spec((1, 2 * C))])
    out_specs = [_row_spec()] * (3 if has_x else 2)
    assert len(in_specs) == n_in
    return pl.pallas_call(
        kern,
        grid=(ROWS // BLK,),
        in_specs=in_specs,
        out_specs=out_specs,
        out_shape=_out_rows(3 if has_x else 2),
    )


@functools.lru_cache(maxsize=None)
def _make_tc2(has_p, has_proj):
    def kern(*refs):
        refs = list(refs)
        p_ref = refs.pop(0) if has_p else None
        r_ref, sr_ref, d_ref, z_ref, h_ref = refs[:5]
        refs = refs[5:]
        wx_ref = refs.pop(0) if has_p else None
        wh_ref, b_ref = refs.pop(0), refs.pop(0)
        if has_proj:
            wp_ref, bp_ref = refs.pop(0), refs.pop(0)
        hn_ref = refs.pop(0)
        xn_ref = refs.pop(0) if has_proj else None
        dg = _dg(d_ref)
        g = r_ref[...] + dg * sr_ref[...]
        acc = jnp.dot(g, wh_ref[...], preferred_element_type=jnp.float32)
        if has_p:
            acc += jnp.dot(p_ref[...], wx_ref[...],
                           preferred_element_type=jnp.float32)
        ht = jnp.tanh(acc + b_ref[...])
        z = z_ref[...]
        hn = z * h_ref[...] + (1.0 - z) * ht
        hn_ref[...] = hn
        if has_proj:
            xn_ref[...] = (jnp.dot(hn, wp_ref[...],
                                   preferred_element_type=jnp.float32)
                           + bp_ref[...])

    in_specs = ([_row_spec()] * (3 if has_p else 2) + [_deg_spec()]
                + [_row_spec()] * 2
                + [_w_spec((C, C))] * (2 if has_p else 1)
                + [_w_spec((1, C))])
    if has_proj:
        in_specs += [_w_spec((C, C)), _w_spec((1, C))]
    out_specs = [_row_spec()] * (2 if has_proj else 1)
    return pl.pallas_call(
        kern,
        grid=(ROWS // BLK,),
        in_specs=in_specs,
        out_specs=out_specs,
        out_shape=_out_rows(2 if has_proj else 1),
    )


def _seg(planes, src3, dst3, zrow):
    u = jnp.concatenate([p.reshape(B, N, C) for p in planes], axis=0)
    s = _make_seg(u.shape[0])(u, src3, dst3, zrow)
    return [s[i * B:(i + 1) * B, :N].reshape(ROWS, C)
            for i in range(len(planes))]


def kernel(encoder_hidden_state, edge_index, W_zr0, b_zr0, W_h0, b_h0,
           W_zr1, b_zr1, W_h1, b_h1, W_proj, b_proj):
    src = edge_index[0]
    dst = edge_index[1]
    order = jnp.argsort(src)
    src = src[order]
    dst = dst[order]
    pad = EPAD - E
    src3 = jnp.concatenate([src, jnp.zeros((pad,), jnp.int32)]
                           ).reshape(NTILES, NCH, CHUNK)
    dst3 = jnp.concatenate([dst, jnp.full((pad,), N, jnp.int32)]
                           ).reshape(NTILES, NCH, CHUNK)
    zrow = jnp.zeros((CHUNK, C), jnp.float32)

    (degfull,) = _seg([jnp.ones((ROWS, C), jnp.float32)], src3, dst3, zrow)
    deg16 = degfull[:N, :16]

    h0 = encoder_hidden_state[0].reshape(ROWS, C)
    h1 = encoder_hidden_state[1].reshape(ROWS, C)

    wzr0_x, wzr0_h = W_zr0[:OUT], W_zr0[OUT:]
    wh0_x, wh0_h = W_h0[:OUT], W_h0[OUT:]
    wzr1_x, wzr1_h = W_zr1[:C], W_zr1[C:]
    wh1_x, wh1_h = W_h1[:C], W_h1[C:]
    bzr0 = b_zr0.reshape(1, 2 * C)
    bh0 = b_h0.reshape(1, C)
    bzr1 = b_zr1.reshape(1, 2 * C)
    bh1 = b_h1.reshape(1, C)
    wp = W_proj
    bp = b_proj.reshape(1, C)

    x = None
    outs = []
    for _ in range(HORIZON):
        if x is None:
            (sh,) = _seg([h0], src3, dst3, zrow)
            z, r = _make_tc1(False)(h0, sh, deg16, wzr0_h, bzr0)
            (sr,) = _seg([r], src3, dst3, zrow)
            (h0,) = _make_tc2(False, False)(r, sr, deg16, z, h0, wh0_h, bh0)
        else:
            sx, sh = _seg([x, h0], src3, dst3, zrow)
            p, z, r = _make_tc1(True)(x, h0, sx, sh, deg16, wzr0_x, wzr0_h,
                                      bzr0)
            (sr,) = _seg([r], src3, dst3, zrow)
            (h0,) = _make_tc2(True, False)(p, r, sr, deg16, z, h0, wh0_x,
                                           wh0_h, bh0)
        sx, sh = _seg([h0, h1], src3, dst3, zrow)
        p, z, r = _make_tc1(True)(h0, h1, sx, sh, deg16, wzr1_x, wzr1_h, bzr1)
        (sr,) = _seg([r], src3, dst3, zrow)
        h1, x = _make_tc2(True, True)(p, r, sr, deg16, z, h1, wh1_x, wh1_h,
                                      bh1, wp, bp)
        outs.append(x.reshape(B, N, OUT))
    return jnp.stack(outs)


# speedup vs baseline: 1.1124x; 1.1124x over previous
import functools

import jax
import jax.numpy as jnp
import numpy as np
from jax import lax
from jax.experimental import pallas as pl
from jax.experimental.pallas import tpu as pltpu
from jax.experimental.pallas import tpu_sc as plsc

N = 10000
E = 160000
C = 128
OUT = 128
B = 2
HORIZON = 3

NTILES = 16
CHUNK = 128
NCH = 80
EPT = NCH * CHUNK
EPAD = NTILES * EPT
NPAD = 10240
RPT = NPAD // NTILES
HALF = NCH // 5
NBUF = 2

ROWS = B * N
BLK = 2000
NPB = N // BLK


def _seg_body(PP, u_hbm, src_hbm, dst_hbm, zrow_hbm, out_hbm,
              acc, srcv, dstv, bufs, gsems, ssems):
    c = lax.axis_index("c")
    w = lax.axis_index("s")

    def wait_g(u, k, b):
        pltpu.make_async_copy(u.at[srcv.at[k]], bufs[b], gsems[b]).wait()

    def start_g(u, k, b):
        pltpu.async_copy(u.at[srcv.at[k]], bufs[b], gsems[b])

    def start_s(k, b):
        pltpu.async_copy(bufs[b], acc.at[dstv.at[k]], ssems[b], add=True)

    def wait_s(k, b):
        pltpu.make_async_copy(bufs[b], acc.at[dstv.at[k]], ssems[b]).wait()

    for j in range(PP):
        plane = c * PP + j
        u = u_hbm.at[plane]
        pltpu.sync_copy(zrow_hbm, bufs[0])
        for q in range(RPT // CHUNK):
            pltpu.sync_copy(bufs[0], acc.at[pl.ds(w * RPT + q * CHUNK, CHUNK)])
        plsc.subcore_barrier()
        for h in range(NCH // HALF):
            pltpu.sync_copy(src_hbm.at[w].at[pl.ds(h * HALF, HALF)], srcv)
            pltpu.sync_copy(dst_hbm.at[w].at[pl.ds(h * HALF, HALF)], dstv)
            start_g(u, 0, 0)
            start_g(u, 1, 1)
            wait_g(u, 0, 0)
            start_s(0, 0)
            def body(i, _):
                for t in range(NBUF):
                    k = NBUF * i + 1 + t
                    b = (1 + t) % NBUF
                    wait_g(u, k, b)
                    start_s(k, b)
                    wait_s(k - 1, (b + 1) % NBUF)
                    start_g(u, k + 1, (b + 1) % NBUF)
                return 0

            lax.fori_loop(0, (HALF - 2) // NBUF, body, 0)
            wait_g(u, HALF - 1, (HALF - 1) % NBUF)
            start_s(HALF - 1, (HALF - 1) % NBUF)
            for k in (HALF - 2, HALF - 1):
                wait_s(k, k % NBUF)
        plsc.subcore_barrier()
        pltpu.sync_copy(acc.at[pl.ds(w * RPT, RPT)],
                        out_hbm.at[plane].at[pl.ds(w * RPT, RPT)])


@functools.lru_cache(maxsize=None)
def _make_seg(P):
    PP = P // 2
    mesh = plsc.VectorSubcoreMesh(core_axis_name="c", subcore_axis_name="s")

    @functools.partial(
        pl.kernel, mesh=mesh,
        out_type=jax.ShapeDtypeStruct((P, NPAD, C), jnp.float32),
        scratch_types=(
            [pltpu.VMEM_SHARED((NPAD, C), jnp.float32)]
            + [pltpu.VMEM((HALF, CHUNK), jnp.int32)] * 2
            + [pltpu.VMEM((CHUNK, C), jnp.float32)] * NBUF
            + [pltpu.SemaphoreType.DMA] * (2 * NBUF)
        ),
    )
    def seg(u_hbm, src_hbm, dst_hbm, zrow_hbm, out_hbm, *rest):
        acc, srcv, dstv = rest[0], rest[1], rest[2]
        bufs = rest[3:3 + NBUF]
        gsems = rest[3 + NBUF:3 + 2 * NBUF]
        ssems = rest[3 + 2 * NBUF:3 + 3 * NBUF]
        _seg_body(PP, u_hbm, src_hbm, dst_hbm, zrow_hbm, out_hbm,
                  acc, srcv, dstv, bufs, gsems, ssems)

    return seg


def _dg(d_ref):
    return 1.0 / jnp.maximum(d_ref[:, 0:1], 1.0)


def _row_spec():
    return pl.BlockSpec((BLK, C), lambda i: (i, 0))


def _deg_spec():
    return pl.BlockSpec((BLK, 16), lambda i: (i % NPB, 0))


def _w_spec(shape):
    return pl.BlockSpec(shape, lambda i: (0, 0))


def _out_rows(n):
    return [jax.ShapeDtypeStruct((ROWS, C), jnp.float32) for _ in range(n)]


@functools.lru_cache(maxsize=None)
def _make_tc1(has_x):
    def kern(*refs):
        if has_x:
            (x_ref, h_ref, sx_ref, sh_ref, d_ref, wx_ref, wh_ref, b_ref,
             p_ref, z_ref, r_ref) = refs
        else:
            (h_ref, sh_ref, d_ref, wh_ref, b_ref, z_ref, r_ref) = refs
        dg = _dg(d_ref)
        q = h_ref[...] + dg * sh_ref[...]
        acc = jnp.dot(q, wh_ref[...], preferred_element_type=jnp.float32)
        if has_x:
            p = x_ref[...] + dg * sx_ref[...]
            acc += jnp.dot(p, wx_ref[...], preferred_element_type=jnp.float32)
            p_ref[...] = p
        zr = jax.nn.sigmoid(acc + b_ref[...])
        z_ref[...] = zr[:, :C]
        r_ref[...] = zr[:, C:] * h_ref[...]

    n_in = 8 if has_x else 5
    in_specs = ([_row_spec()] * (4 if has_x else 2) + [_deg_spec()]
                + [_w_spec((C, 2 * C))] * (2 if has_x else 1)
                + [_w_spec((1, 2 * C))])
    out_specs = [_row_spec()] * (3 if has_x else 2)
    assert len(in_specs) == n_in
    return pl.pallas_call(
        kern,
        grid=(ROWS // BLK,),
        in_specs=in_specs,
        out_specs=out_specs,
        out_shape=_out_rows(3 if has_x else 2),
    )


@functools.lru_cache(maxsize=None)
def _make_tc2(has_p, has_proj):
    def kern(*refs):
        refs = list(refs)
        p_ref = refs.pop(0) if has_p else None
        r_ref, sr_ref, d_ref, z_ref, h_ref = refs[:5]
        refs = refs[5:]
        wx_ref = refs.pop(0) if has_p else None
        wh_ref, b_ref = refs.pop(0), refs.pop(0)
        if has_proj:
            wp_ref, bp_ref = refs.pop(0), refs.pop(0)
        hn_ref = refs.pop(0)
        xn_ref = refs.pop(0) if has_proj else None
        dg = _dg(d_ref)
        g = r_ref[...] + dg * sr_ref[...]
        acc = jnp.dot(g, wh_ref[...], preferred_element_type=jnp.float32)
        if has_p:
            acc += jnp.dot(p_ref[...], wx_ref[...],
                           preferred_element_type=jnp.float32)
        ht = jnp.tanh(acc + b_ref[...])
        z = z_ref[...]
        hn = z * h_ref[...] + (1.0 - z) * ht
        hn_ref[...] = hn
        if has_proj:
            xn_ref[...] = (jnp.dot(hn, wp_ref[...],
                                   preferred_element_type=jnp.float32)
                           + bp_ref[...])

    in_specs = ([_row_spec()] * (3 if has_p else 2) + [_deg_spec()]
                + [_row_spec()] * 2
                + [_w_spec((C, C))] * (2 if has_p else 1)
                + [_w_spec((1, C))])
    if has_proj:
        in_specs += [_w_spec((C, C)), _w_spec((1, C))]
    out_specs = [_row_spec()] * (2 if has_proj else 1)
    return pl.pallas_call(
        kern,
        grid=(ROWS // BLK,),
        in_specs=in_specs,
        out_specs=out_specs,
        out_shape=_out_rows(2 if has_proj else 1),
    )


def _seg(planes, src3, dst3, zrow):
    u = jnp.concatenate([p.reshape(B, N, C) for p in planes], axis=0)
    s = _make_seg(u.shape[0])(u, src3, dst3, zrow)
    return [s[i * B:(i + 1) * B, :N].reshape(ROWS, C)
            for i in range(len(planes))]


def kernel(encoder_hidden_state, edge_index, W_zr0, b_zr0, W_h0, b_h0,
           W_zr1, b_zr1, W_h1, b_h1, W_proj, b_proj):
    src = edge_index[0]
    dst = edge_index[1]
    pad = EPAD - E
    src3 = jnp.concatenate([src, jnp.zeros((pad,), jnp.int32)]
                           ).reshape(NTILES, NCH, CHUNK)
    dst3 = jnp.concatenate([dst, jnp.full((pad,), N, jnp.int32)]
                           ).reshape(NTILES, NCH, CHUNK)
    zrow = jnp.zeros((CHUNK, C), jnp.float32)

    (degfull,) = _seg([jnp.ones((ROWS, C), jnp.float32)], src3, dst3, zrow)
    deg16 = degfull[:N, :16]

    h0 = encoder_hidden_state[0].reshape(ROWS, C)
    h1 = encoder_hidden_state[1].reshape(ROWS, C)

    wzr0_x, wzr0_h = W_zr0[:OUT], W_zr0[OUT:]
    wh0_x, wh0_h = W_h0[:OUT], W_h0[OUT:]
    wzr1_x, wzr1_h = W_zr1[:C], W_zr1[C:]
    wh1_x, wh1_h = W_h1[:C], W_h1[C:]
    bzr0 = b_zr0.reshape(1, 2 * C)
    bh0 = b_h0.reshape(1, C)
    bzr1 = b_zr1.reshape(1, 2 * C)
    bh1 = b_h1.reshape(1, C)
    wp = W_proj
    bp = b_proj.reshape(1, C)

    x = None
    outs = []
    for _ in range(HORIZON):
        if x is None:
            (sh,) = _seg([h0], src3, dst3, zrow)
            z, r = _make_tc1(False)(h0, sh, deg16, wzr0_h, bzr0)
            (sr,) = _seg([r], src3, dst3, zrow)
            (h0,) = _make_tc2(False, False)(r, sr, deg16, z, h0, wh0_h, bh0)
        else:
            sx, sh = _seg([x, h0], src3, dst3, zrow)
            p, z, r = _make_tc1(True)(x, h0, sx, sh, deg16, wzr0_x, wzr0_h,
                                      bzr0)
            (sr,) = _seg([r], src3, dst3, zrow)
            (h0,) = _make_tc2(True, False)(p, r, sr, deg16, z, h0, wh0_x,
                                           wh0_h, bh0)
        sx, sh = _seg([h0, h1], src3, dst3, zrow)
        p, z, r = _make_tc1(True)(h0, h1, sx, sh, deg16, wzr1_x, wzr1_h, bzr1)
        (sr,) = _seg([r], src3, dst3, zrow)
        h1, x = _make_tc2(True, True)(p, r, sr, deg16, z, h1, wh1_x, wh1_h,
                                      bh1, wp, bp)
        outs.append(x.reshape(B, N, OUT))
    return jnp.stack(outs)


# speedup vs baseline: 1.1469x; 1.0310x over previous
import functools

import jax
import jax.numpy as jnp
import numpy as np
from jax import lax
from jax.experimental import pallas as pl
from jax.experimental.pallas import tpu as pltpu
from jax.experimental.pallas import tpu_sc as plsc

N = 10000
E = 160000
C = 128
OUT = 128
B = 2
HORIZON = 3

NTILES = 16
CHUNK = 64
NCH = 160
EPT = NCH * CHUNK
EPAD = NTILES * EPT
NPAD = 10240
RPT = NPAD // NTILES
HALF = NCH // 4
NBUF = 4

ROWS = B * N
BLK = 2000
NPB = N // BLK


def _seg_body(PP, u_hbm, src_hbm, dst_hbm, zrow_hbm, out_hbm,
              acc, srcv, dstv, bufs, gsems, ssems):
    c = lax.axis_index("c")
    w = lax.axis_index("s")

    def wait_g(u, k, b):
        pltpu.make_async_copy(u.at[srcv.at[k]], bufs[b], gsems[b]).wait()

    def start_g(u, k, b):
        pltpu.async_copy(u.at[srcv.at[k]], bufs[b], gsems[b])

    def start_s(k, b):
        pltpu.async_copy(bufs[b], acc.at[dstv.at[k]], ssems[b], add=True)

    def wait_s(k, b):
        pltpu.make_async_copy(bufs[b], acc.at[dstv.at[k]], ssems[b]).wait()

    for j in range(PP):
        plane = c * PP + j
        u = u_hbm.at[plane]
        pltpu.sync_copy(zrow_hbm, bufs[0])
        for q in range(RPT // CHUNK):
            pltpu.sync_copy(bufs[0], acc.at[pl.ds(w * RPT + q * CHUNK, CHUNK)])
        plsc.subcore_barrier()
        for h in range(NCH // HALF):
            pltpu.sync_copy(src_hbm.at[w].at[pl.ds(h * HALF, HALF)], srcv)
            pltpu.sync_copy(dst_hbm.at[w].at[pl.ds(h * HALF, HALF)], dstv)
            for b in range(NBUF):
                start_g(u, b, b)
            for k in (0, 1):
                wait_g(u, k, k)
                start_s(k, k)
            def body(i, _):
                for t in range(NBUF):
                    k = NBUF * i + 2 + t
                    b = (2 + t) % NBUF
                    wait_g(u, k, b)
                    start_s(k, b)
                    wait_s(k - 2, (b + 2) % NBUF)
                    start_g(u, k + 2, (b + 2) % NBUF)
                return 0

            lax.fori_loop(0, (HALF - 4) // NBUF, body, 0)
            for k in (HALF - 2, HALF - 1):
                b = k % NBUF
                wait_g(u, k, b)
                start_s(k, b)
            for k in range(HALF - 4, HALF):
                wait_s(k, k % NBUF)
        plsc.subcore_barrier()
        pltpu.sync_copy(acc.at[pl.ds(w * RPT, RPT)],
                        out_hbm.at[plane].at[pl.ds(w * RPT, RPT)])


@functools.lru_cache(maxsize=None)
def _make_seg(P):
    PP = P // 2
    mesh = plsc.VectorSubcoreMesh(core_axis_name="c", subcore_axis_name="s")

    @functools.partial(
        pl.kernel, mesh=mesh,
        out_type=jax.ShapeDtypeStruct((P, NPAD, C), jnp.float32),
        scratch_types=(
            [pltpu.VMEM_SHARED((NPAD, C), jnp.float32)]
            + [pltpu.VMEM((HALF, CHUNK), jnp.int32)] * 2
            + [pltpu.VMEM((CHUNK, C), jnp.float32)] * NBUF
            + [pltpu.SemaphoreType.DMA] * (2 * NBUF)
        ),
    )
    def seg(u_hbm, src_hbm, dst_hbm, zrow_hbm, out_hbm, *rest):
        acc, srcv, dstv = rest[0], rest[1], rest[2]
        bufs = rest[3:3 + NBUF]
        gsems = rest[3 + NBUF:3 + 2 * NBUF]
        ssems = rest[3 + 2 * NBUF:3 + 3 * NBUF]
        _seg_body(PP, u_hbm, src_hbm, dst_hbm, zrow_hbm, out_hbm,
                  acc, srcv, dstv, bufs, gsems, ssems)

    return seg


def _dg(d_ref):
    return 1.0 / jnp.maximum(d_ref[:, 0:1], 1.0)


def _row_spec():
    return pl.BlockSpec((BLK, C), lambda i: (i, 0))


def _deg_spec():
    return pl.BlockSpec((BLK, 16), lambda i: (i % NPB, 0))


def _w_spec(shape):
    return pl.BlockSpec(shape, lambda i: (0, 0))


def _out_rows(n):
    return [jax.ShapeDtypeStruct((ROWS, C), jnp.float32) for _ in range(n)]


@functools.lru_cache(maxsize=None)
def _make_tc1(has_x):
    def kern(*refs):
        if has_x:
            (x_ref, h_ref, sx_ref, sh_ref, d_ref, wx_ref, wh_ref, b_ref,
             p_ref, z_ref, r_ref) = refs
        else:
            (h_ref, sh_ref, d_ref, wh_ref, b_ref, z_ref, r_ref) = refs
        dg = _dg(d_ref)
        q = h_ref[...] + dg * sh_ref[...]
        acc = jnp.dot(q, wh_ref[...], preferred_element_type=jnp.float32)
        if has_x:
            p = x_ref[...] + dg * sx_ref[...]
            acc += jnp.dot(p, wx_ref[...], preferred_element_type=jnp.float32)
            p_ref[...] = p
        zr = jax.nn.sigmoid(acc + b_ref[...])
        z_ref[...] = zr[:, :C]
        r_ref[...] = zr[:, C:] * h_ref[...]

    n_in = 8 if has_x else 5
    in_specs = ([_row_spec()] * (4 if has_x else 2) + [_deg_spec()]
                + [_w_spec((C, 2 * C))] * (2 if has_x else 1)
                + [_w_spec((1, 2 * C))])
    out_specs = [_row_spec()] * (3 if has_x else 2)
    assert len(in_specs) == n_in
    return pl.pallas_call(
        kern,
        grid=(ROWS // BLK,),
        in_specs=in_specs,
        out_specs=out_specs,
        out_shape=_out_rows(3 if has_x else 2),
    )


@functools.lru_cache(maxsize=None)
def _make_tc2(has_p, has_proj):
    def kern(*refs):
        refs = list(refs)
        p_ref = refs.pop(0) if has_p else None
        r_ref, sr_ref, d_ref, z_ref, h_ref = refs[:5]
        refs = refs[5:]
        wx_ref = refs.pop(0) if has_p else None
        wh_ref, b_ref = refs.pop(0), refs.pop(0)
        if has_proj:
            wp_ref, bp_ref = refs.pop(0), refs.pop(0)
        hn_ref = refs.pop(0)
        xn_ref = refs.pop(0) if has_proj else None
        dg = _dg(d_ref)
        g = r_ref[...] + dg * sr_ref[...]
        acc = jnp.dot(g, wh_ref[...], preferred_element_type=jnp.float32)
        if has_p:
            acc += jnp.dot(p_ref[...], wx_ref[...],
                           preferred_element_type=jnp.float32)
        ht = jnp.tanh(acc + b_ref[...])
        z = z_ref[...]
        hn = z * h_ref[...] + (1.0 - z) * ht
        hn_ref[...] = hn
        if has_proj:
            xn_ref[...] = (jnp.dot(hn, wp_ref[...],
                                   preferred_element_type=jnp.float32)
                           + bp_ref[...])

    in_specs = ([_row_spec()] * (3 if has_p else 2) + [_deg_spec()]
                + [_row_spec()] * 2
                + [_w_spec((C, C))] * (2 if has_p else 1)
                + [_w_spec((1, C))])
    if has_proj:
        in_specs += [_w_spec((C, C)), _w_spec((1, C))]
    out_specs = [_row_spec()] * (2 if has_proj else 1)
    return pl.pallas_call(
        kern,
        grid=(ROWS // BLK,),
        in_specs=in_specs,
        out_specs=out_specs,
        out_shape=_out_rows(2 if has_proj else 1),
    )


def _seg(planes, src3, dst3, zrow):
    u = jnp.concatenate([p.reshape(B, N, C) for p in planes], axis=0)
    s = _make_seg(u.shape[0])(u, src3, dst3, zrow)
    return [s[i * B:(i + 1) * B, :N].reshape(ROWS, C)
            for i in range(len(planes))]


def kernel(encoder_hidden_state, edge_index, W_zr0, b_zr0, W_h0, b_h0,
           W_zr1, b_zr1, W_h1, b_h1, W_proj, b_proj):
    src = edge_index[0]
    dst = edge_index[1]
    pad = EPAD - E
    src3 = jnp.concatenate([src, jnp.zeros((pad,), jnp.int32)]
                           ).reshape(NTILES, NCH, CHUNK)
    dst3 = jnp.concatenate([dst, jnp.full((pad,), N, jnp.int32)]
                           ).reshape(NTILES, NCH, CHUNK)
    zrow = jnp.zeros((CHUNK, C), jnp.float32)

    h0 = encoder_hidden_state[0].reshape(ROWS, C)
    h1 = encoder_hidden_state[1].reshape(ROWS, C)

    wzr0_x, wzr0_h = W_zr0[:OUT], W_zr0[OUT:]
    wh0_x, wh0_h = W_h0[:OUT], W_h0[OUT:]
    wzr1_x, wzr1_h = W_zr1[:C], W_zr1[C:]
    wh1_x, wh1_h = W_h1[:C], W_h1[C:]
    bzr0 = b_zr0.reshape(1, 2 * C)
    bh0 = b_h0.reshape(1, C)
    bzr1 = b_zr1.reshape(1, 2 * C)
    bh1 = b_h1.reshape(1, C)
    wp = W_proj
    bp = b_proj.reshape(1, C)

    x = None
    outs = []
    for _ in range(HORIZON):
        if x is None:
            sh, degrows = _seg([h0, jnp.ones((ROWS, C), jnp.float32)],
                               src3, dst3, zrow)
            deg16 = degrows[:N, :16]
            z, r = _make_tc1(False)(h0, sh, deg16, wzr0_h, bzr0)
            (sr,) = _seg([r], src3, dst3, zrow)
            (h0,) = _make_tc2(False, False)(r, sr, deg16, z, h0, wh0_h, bh0)
        else:
            sx, sh = _seg([x, h0], src3, dst3, zrow)
            p, z, r = _make_tc1(True)(x, h0, sx, sh, deg16, wzr0_x, wzr0_h,
                                      bzr0)
            (sr,) = _seg([r], src3, dst3, zrow)
            (h0,) = _make_tc2(True, False)(p, r, sr, deg16, z, h0, wh0_x,
                                           wh0_h, bh0)
        sx, sh = _seg([h0, h1], src3, dst3, zrow)
        p, z, r = _make_tc1(True)(h0, h1, sx, sh, deg16, wzr1_x, wzr1_h, bzr1)
        (sr,) = _seg([r], src3, dst3, zrow)
        h1, x = _make_tc2(True, True)(p, r, sr, deg16, z, h1, wh1_x, wh1_h,
                                      bh1, wp, bp)
        outs.append(x.reshape(B, N, OUT))
    return jnp.stack(outs)


# speedup vs baseline: 1.2213x; 1.0648x over previous
import functools

import jax
import jax.numpy as jnp
from jax import lax
from jax.experimental import pallas as pl
from jax.experimental.pallas import tpu as pltpu
from jax.experimental.pallas import tpu_sc as plsc

N = 10000
E = 160000
C = 128
OUT = 128
B = 2
HORIZON = 3

NTILES = 16
CHUNK = 64
NCH = 160
EPT = NCH * CHUNK
EPAD = NTILES * EPT
NPAD = 10240
RPT = NPAD // NTILES
HALF = NCH // 4
NBUF = 4

ROWS = B * N
BLK = 2000
NPB = N // BLK


def _seg_body(PP, u_hbm, src_hbm, dst_hbm, zrow_hbm, out_hbm,
              acc, srcv, dstv, bufs, gsems, ssems):
    c = lax.axis_index("c")
    w = lax.axis_index("s")

    def wait_g(u, k, b):
        pltpu.make_async_copy(u.at[srcv.at[k]], bufs[b], gsems[b]).wait()

    def start_g(u, k, b):
        pltpu.async_copy(u.at[srcv.at[k]], bufs[b], gsems[b])

    def start_s(k, b):
        pltpu.async_copy(bufs[b], acc.at[dstv.at[k]], ssems[b], add=True)

    def wait_s(k, b):
        pltpu.make_async_copy(bufs[b], acc.at[dstv.at[k]], ssems[b]).wait()

    for j in range(PP):
        plane = c * PP + j
        u = u_hbm.at[plane]
        pltpu.sync_copy(zrow_hbm, bufs[0])
        for q in range(RPT // CHUNK):
            pltpu.sync_copy(bufs[0], acc.at[pl.ds(w * RPT + q * CHUNK, CHUNK)])
        plsc.subcore_barrier()
        for h in range(NCH // HALF):
            pltpu.sync_copy(src_hbm.at[w].at[pl.ds(h * HALF, HALF)], srcv)
            pltpu.sync_copy(dst_hbm.at[w].at[pl.ds(h * HALF, HALF)], dstv)
            for b in range(NBUF):
                start_g(u, b, b)
            for k in (0, 1):
                wait_g(u, k, k)
                start_s(k, k)
            def body(i, _):
                for t in range(NBUF):
                    k = NBUF * i + 2 + t
                    b = (2 + t) % NBUF
                    wait_g(u, k, b)
                    start_s(k, b)
                    wait_s(k - 2, (b + 2) % NBUF)
                    start_g(u, k + 2, (b + 2) % NBUF)
                return 0

            lax.fori_loop(0, (HALF - 4) // NBUF, body, 0)
            for k in (HALF - 2, HALF - 1):
                b = k % NBUF
                wait_g(u, k, b)
                start_s(k, b)
            for k in range(HALF - 4, HALF):
                wait_s(k, k % NBUF)
        plsc.subcore_barrier()
        pltpu.sync_copy(acc.at[pl.ds(w * RPT, RPT)],
                        out_hbm.at[plane].at[pl.ds(w * RPT, RPT)])


@functools.lru_cache(maxsize=None)
def _make_seg(P):
    PP = P // 2
    mesh = plsc.VectorSubcoreMesh(core_axis_name="c", subcore_axis_name="s")

    @functools.partial(
        pl.kernel, mesh=mesh,
        out_type=jax.ShapeDtypeStruct((P, NPAD, C), jnp.float32),
        scratch_types=(
            [pltpu.VMEM_SHARED((NPAD, C), jnp.float32)]
            + [pltpu.VMEM((HALF, CHUNK), jnp.int32)] * 2
            + [pltpu.VMEM((CHUNK, C), jnp.float32)] * NBUF
            + [pltpu.SemaphoreType.DMA] * (2 * NBUF)
        ),
    )
    def seg(u_hbm, src_hbm, dst_hbm, zrow_hbm, out_hbm, *rest):
        acc, srcv, dstv = rest[0], rest[1], rest[2]
        bufs = rest[3:3 + NBUF]
        gsems = rest[3 + NBUF:3 + 2 * NBUF]
        ssems = rest[3 + 2 * NBUF:3 + 3 * NBUF]
        _seg_body(PP, u_hbm, src_hbm, dst_hbm, zrow_hbm, out_hbm,
                  acc, srcv, dstv, bufs, gsems, ssems)

    return seg


def _dg(d_ref):
    return 1.0 / jnp.maximum(d_ref[:, 0:1], 1.0)


def _row_spec():
    return pl.BlockSpec((BLK, C), lambda i: (i, 0))


def _deg_spec():
    return pl.BlockSpec((BLK, 16), lambda i: (i % NPB, 0))


def _w_spec(shape):
    return pl.BlockSpec(shape, lambda i: (0, 0))


def _out_rows(n):
    return [jax.ShapeDtypeStruct((ROWS, C), jnp.float32) for _ in range(n)]


@functools.lru_cache(maxsize=None)
def _make_tc1(has_x):
    def kern(*refs):
        if has_x:
            (x_ref, h_ref, sx_ref, sh_ref, d_ref, wx_ref, wh_ref, b_ref,
             p_ref, z_ref, r_ref) = refs
        else:
            (h_ref, sh_ref, d_ref, wh_ref, b_ref, z_ref, r_ref) = refs
        dg = _dg(d_ref)
        q = h_ref[...] + dg * sh_ref[...]
        acc = jnp.dot(q, wh_ref[...], preferred_element_type=jnp.float32)
        if has_x:
            p = x_ref[...] + dg * sx_ref[...]
            acc += jnp.dot(p, wx_ref[...], preferred_element_type=jnp.float32)
            p_ref[...] = p
        zr = jax.nn.sigmoid(acc + b_ref[...])
        z_ref[...] = zr[:, :C]
        r_ref[...] = zr[:, C:] * h_ref[...]

    n_in = 8 if has_x else 5
    in_specs = ([_row_spec()] * (4 if has_x else 2) + [_deg_spec()]
                + [_w_spec((C, 2 * C))] * (2 if has_x else 1)
                + [_w_spec((1, 2 * C))])
    out_specs = [_row_spec()] * (3 if has_x else 2)
    assert len(in_specs) == n_in
    return pl.pallas_call(
        kern,
        grid=(ROWS // BLK,),
        in_specs=in_specs,
        out_specs=out_specs,
        out_shape=_out_rows(3 if has_x else 2),
    )


@functools.lru_cache(maxsize=None)
def _make_tc2(has_p, has_proj):
    def kern(*refs):
        refs = list(refs)
        p_ref = refs.pop(0) if has_p else None
        r_ref, sr_ref, d_ref, z_ref, h_ref = refs[:5]
        refs = refs[5:]
        wx_ref = refs.pop(0) if has_p else None
        wh_ref, b_ref = refs.pop(0), refs.pop(0)
        if has_proj:
            wp_ref, bp_ref = refs.pop(0), refs.pop(0)
        hn_ref = refs.pop(0)
        xn_ref = refs.pop(0) if has_proj else None
        dg = _dg(d_ref)
        g = r_ref[...] + dg * sr_ref[...]
        acc = jnp.dot(g, wh_ref[...], preferred_element_type=jnp.float32)
        if has_p:
            acc += jnp.dot(p_ref[...], wx_ref[...],
                           preferred_element_type=jnp.float32)
        ht = jnp.tanh(acc + b_ref[...])
        z = z_ref[...]
        hn = z * h_ref[...] + (1.0 - z) * ht
        hn_ref[...] = hn
        if has_proj:
            xn_ref[...] = (jnp.dot(hn, wp_ref[...],
                                   preferred_element_type=jnp.float32)
                           + bp_ref[...])

    in_specs = ([_row_spec()] * (3 if has_p else 2) + [_deg_spec()]
                + [_row_spec()] * 2
                + [_w_spec((C, C))] * (2 if has_p else 1)
                + [_w_spec((1, C))])
    if has_proj:
        in_specs += [_w_spec((C, C)), _w_spec((1, C))]
    out_specs = [_row_spec()] * (2 if has_proj else 1)
    return pl.pallas_call(
        kern,
        grid=(ROWS // BLK,),
        in_specs=in_specs,
        out_specs=out_specs,
        out_shape=_out_rows(2 if has_proj else 1),
    )


def _seg(planes, src3, dst3, zrow):
    u = jnp.concatenate([p.reshape(B, N, C) for p in planes], axis=0)
    s = _make_seg(u.shape[0])(u, src3, dst3, zrow)
    return [s[i * B:(i + 1) * B, :N].reshape(ROWS, C)
            for i in range(len(planes))]


def kernel(encoder_hidden_state, edge_index, W_zr0, b_zr0, W_h0, b_h0,
           W_zr1, b_zr1, W_h1, b_h1, W_proj, b_proj):
    src = edge_index[0]
    dst = edge_index[1]
    pad = EPAD - E
    src3 = jnp.concatenate([src, jnp.zeros((pad,), jnp.int32)]
                           ).reshape(NTILES, NCH, CHUNK)
    dst3 = jnp.concatenate([dst, jnp.full((pad,), N, jnp.int32)]
                           ).reshape(NTILES, NCH, CHUNK)
    zrow = jnp.zeros((CHUNK, C), jnp.float32)

    (degfull,) = _seg([jnp.ones((ROWS, C), jnp.float32)], src3, dst3, zrow)
    deg16 = degfull[:N, :16]

    h0 = encoder_hidden_state[0].reshape(ROWS, C)
    h1 = encoder_hidden_state[1].reshape(ROWS, C)

    wzr0_x, wzr0_h = W_zr0[:OUT], W_zr0[OUT:]
    wh0_x, wh0_h = W_h0[:OUT], W_h0[OUT:]
    wzr1_x, wzr1_h = W_zr1[:C], W_zr1[C:]
    wh1_x, wh1_h = W_h1[:C], W_h1[C:]
    bzr0 = b_zr0.reshape(1, 2 * C)
    bh0 = b_h0.reshape(1, C)
    bzr1 = b_zr1.reshape(1, 2 * C)
    bh1 = b_h1.reshape(1, C)
    wp = W_proj
    bp = b_proj.reshape(1, C)

    x = None
    outs = []
    for _ in range(HORIZON):
        if x is None:
            (sh,) = _seg([h0], src3, dst3, zrow)
            z, r = _make_tc1(False)(h0, sh, deg16, wzr0_h, bzr0)
            (sr,) = _seg([r], src3, dst3, zrow)
            (h0,) = _make_tc2(False, False)(r, sr, deg16, z, h0, wh0_h, bh0)
        else:
            sx, sh = _seg([x, h0], src3, dst3, zrow)
            p, z, r = _make_tc1(True)(x, h0, sx, sh, deg16, wzr0_x, wzr0_h,
                                      bzr0)
            (sr,) = _seg([r], src3, dst3, zrow)
            (h0,) = _make_tc2(True, False)(p, r, sr, deg16, z, h0, wh0_x,
                                           wh0_h, bh0)
        sx, sh = _seg([h0, h1], src3, dst3, zrow)
        p, z, r = _make_tc1(True)(h0, h1, sx, sh, deg16, wzr1_x, wzr1_h, bzr1)
        (sr,) = _seg([r], src3, dst3, zrow)
        h1, x = _make_tc2(True, True)(p, r, sr, deg16, z, h1, wh1_x, wh1_h,
                                      bh1, wp, bp)
        outs.append(x.reshape(B, N, OUT))
    return jnp.stack(outs)
